# Initial kernel scaffold; baseline (speedup 1.0000x reference)
import functools

import jax
import jax.numpy as jnp
from jax import lax
from jax.experimental import pallas as pl
from jax.experimental.pallas import tpu as pltpu
from jax.experimental.pallas import tpu_sc as plsc

N = 10000
D = 128
H = 4
C = 32
OUT = H * C

NPAD = 10240
NW = 32
EB = 128
WREC = OUT + 16
ROWS_PER_SUB = NPAD // 16


def _proj_body(x_ref, wl_ref, bl_ref, wr_ref, br_ref, xl_ref, xr_ref):
    xb = x_ref[...]
    xl_ref[...] = (
        jnp.dot(xb, wl_ref[...], preferred_element_type=jnp.float32) + bl_ref[...]
    )
    xr_ref[...] = (
        jnp.dot(xb, wr_ref[...], preferred_element_type=jnp.float32) + br_ref[...]
    )


def _project(x_pad, Wl, bl, Wr, br):
    blk = 512
    grid = NPAD // blk
    return pl.pallas_call(
        _proj_body,
        grid=(grid,),
        in_specs=[
            pl.BlockSpec((blk, D), lambda i: (i, 0)),
            pl.BlockSpec((D, OUT), lambda i: (0, 0)),
            pl.BlockSpec((1, OUT), lambda i: (0, 0)),
            pl.BlockSpec((D, OUT), lambda i: (0, 0)),
            pl.BlockSpec((1, OUT), lambda i: (0, 0)),
        ],
        out_specs=[
            pl.BlockSpec((blk, OUT), lambda i: (i, 0)),
            pl.BlockSpec((blk, OUT), lambda i: (i, 0)),
        ],
        out_shape=[
            jax.ShapeDtypeStruct((NPAD, OUT), jnp.float32),
            jax.ShapeDtypeStruct((NPAD, OUT), jnp.float32),
        ],
    )(x_pad, Wl, bl.reshape(1, OUT), Wr, br.reshape(1, OUT))


def _make_edge_kernel(chunks_per_worker):
    mesh = plsc.VectorSubcoreMesh(
        core_axis_name="c", subcore_axis_name="s", num_cores=2, num_subcores=16
    )

    @functools.partial(
        pl.kernel,
        out_type=jax.ShapeDtypeStruct((2, NPAD, WREC), jnp.float32),
        mesh=mesh,
        scratch_types=[
            pltpu.VMEM((EB,), jnp.int32),
            pltpu.VMEM((EB,), jnp.int32),
            pltpu.VMEM((EB, OUT), jnp.float32),
            pltpu.VMEM((EB, OUT), jnp.float32),
            pltpu.VMEM((EB, WREC), jnp.float32),
            pltpu.VMEM((8, 16), jnp.float32),
            pltpu.VMEM_SHARED((NPAD, WREC), jnp.float32),
            pltpu.SemaphoreType.DMA,
            pltpu.SemaphoreType.DMA,
        ],
    )
    def edge_kernel(
        xl_hbm, xr_hbm, src_hbm, dst_hbm, att_hbm, out_hbm,
        sidx, didx, xlr, xrr, orow, att_vm, table, sem1, sem2,
    ):
        cid = lax.axis_index("c")
        sid = lax.axis_index("s")
        wid = sid * 2 + cid

        zero16 = jnp.zeros((16,), jnp.float32)

        def zero_row(r, carry):
            for j in range(WREC // 16):
                orow[r, pl.ds(j * 16, 16)] = zero16
            return carry

        lax.fori_loop(0, EB, zero_row, 0)
        for k in range(ROWS_PER_SUB // EB):
            pltpu.sync_copy(orow, table.at[pl.ds(sid * ROWS_PER_SUB + k * EB, EB)])
        plsc.subcore_barrier()

        pltpu.sync_copy(att_hbm, att_vm)
        att_r = [att_vm[k, :] for k in range(8)]
        lane = lax.iota(jnp.int32, 16)

        def chunk_body(j, carry):
            base = (wid * chunks_per_worker + j) * EB
            pltpu.sync_copy(src_hbm.at[pl.ds(base, EB)], sidx)
            pltpu.sync_copy(dst_hbm.at[pl.ds(base, EB)], didx)
            cp1 = pltpu.async_copy(xl_hbm.at[sidx], xlr, sem1)
            cp2 = pltpu.async_copy(xr_hbm.at[didx], xrr, sem2)
            cp1.wait()
            cp2.wait()

            def edge_body(e, ecarry):
                wrow = zero16
                for h in range(H):
                    a0 = xlr[e, pl.ds(h * 32, 16)]
                    a1 = xlr[e, pl.ds(h * 32 + 16, 16)]
                    b0 = xrr[e, pl.ds(h * 32, 16)]
                    b1 = xrr[e, pl.ds(h * 32 + 16, 16)]
                    t0 = a0 + b0
                    t1 = a1 + b1
                    l0 = jnp.maximum(t0, 0.2 * t0)
                    l1 = jnp.maximum(t1, 0.2 * t1)
                    s = l0 * att_r[2 * h] + l1 * att_r[2 * h + 1]
                    sc = jnp.sum(s)
                    wv = jnp.exp(jnp.full((16,), sc, jnp.float32))
                    orow[e, pl.ds(h * 32, 16)] = a0 * wv
                    orow[e, pl.ds(h * 32 + 16, 16)] = a1 * wv
                    wrow = jnp.where(lane == h, wv, wrow)
                orow[e, pl.ds(OUT, 16)] = wrow
                return ecarry

            lax.fori_loop(0, EB, edge_body, 0)
            pltpu.sync_copy(orow, table.at[didx], add=True)
            return carry

        lax.fori_loop(0, chunks_per_worker, chunk_body, 0)
        plsc.subcore_barrier()

        for k in range(ROWS_PER_SUB // EB):
            r0 = sid * ROWS_PER_SUB + k * EB
            pltpu.sync_copy(table.at[pl.ds(r0, EB)], orow)
            pltpu.sync_copy(orow, out_hbm.at[cid, pl.ds(r0, EB)])

    return edge_kernel


def _gate_body(
    num_ref, h_ref, exp_ref, bias_ref, w1a_ref, w1b_ref, b1_ref,
    w2a_ref, w2b_ref, b2_ref, out_ref,
):
    nm = num_ref[0] + num_ref[1]
    agg = nm[:, :OUT]
    den4 = nm[:, OUT:OUT + 4]
    den = jnp.dot(den4, exp_ref[...], preferred_element_type=jnp.float32)
    f = jax.nn.sigmoid(agg / (den + 1e-16) + bias_ref[...])
    hb = h_ref[...]
    ru = jax.nn.sigmoid(
        jnp.dot(f, w1a_ref[...], preferred_element_type=jnp.float32)
        + jnp.dot(hb, w1b_ref[...], preferred_element_type=jnp.float32)
        + b1_ref[...]
    )
    r = ru[:, :OUT]
    u = ru[:, OUT:]
    cc = jnp.tanh(
        jnp.dot(f, w2a_ref[...], preferred_element_type=jnp.float32)
        + jnp.dot(r * hb, w2b_ref[...], preferred_element_type=jnp.float32)
        + b2_ref[...]
    )
    out_ref[...] = u * hb + (1.0 - u) * cc


def _gate(num2, h, bias, W1, b1, W2, b2):
    blk = 500
    grid = N // blk
    expand = (
        jnp.arange(OUT, dtype=jnp.int32)[None, :] // C
        == jnp.arange(4, dtype=jnp.int32)[:, None]
    ).astype(jnp.float32)
    return pl.pallas_call(
        _gate_body,
        grid=(grid,),
        in_specs=[
            pl.BlockSpec((2, blk, WREC), lambda i: (0, i, 0)),
            pl.BlockSpec((blk, OUT), lambda i: (i, 0)),
            pl.BlockSpec((4, OUT), lambda i: (0, 0)),
            pl.BlockSpec((1, OUT), lambda i: (0, 0)),
            pl.BlockSpec((OUT, 2 * OUT), lambda i: (0, 0)),
            pl.BlockSpec((OUT, 2 * OUT), lambda i: (0, 0)),
            pl.BlockSpec((1, 2 * OUT), lambda i: (0, 0)),
            pl.BlockSpec((OUT, OUT), lambda i: (0, 0)),
            pl.BlockSpec((OUT, OUT), lambda i: (0, 0)),
            pl.BlockSpec((1, OUT), lambda i: (0, 0)),
        ],
        out_specs=pl.BlockSpec((blk, OUT), lambda i: (i, 0)),
        out_shape=jax.ShapeDtypeStruct((N, OUT), jnp.float32),
    )(
        num2, h, expand, bias.reshape(1, OUT),
        W1[:OUT], W1[OUT:], b1.reshape(1, 2 * OUT),
        W2[:OUT], W2[OUT:], b2.reshape(1, OUT),
    )


def kernel(x, edge_index, edge_weight, h, Wl, bl, Wr, br, att, bias, W1, b1, W2, b2):
    del edge_weight

    ne = edge_index.shape[1] + N
    ep = ((ne + NW * EB - 1) // (NW * EB)) * (NW * EB)
    chunks_per_worker = ep // (NW * EB)

    loops = jnp.arange(N, dtype=jnp.int32)
    padv = jnp.full((ep - ne,), N, dtype=jnp.int32)
    src = jnp.concatenate([edge_index[0].astype(jnp.int32), loops, padv])
    dst = jnp.concatenate([edge_index[1].astype(jnp.int32), loops, padv])

    x_pad = jnp.pad(x, ((0, NPAD - N), (0, 0)))
    xl, xr = _project(x_pad, Wl, bl, Wr, br)

    att2 = att.reshape(8, 16)
    num2 = _make_edge_kernel(chunks_per_worker)(xl, xr, src, dst, att2)

    return _gate(num2, h, bias, W1, b1, W2, b2)


# speedup vs baseline: 25.5695x; 25.5695x over previous
import functools

import jax
import jax.numpy as jnp
from jax import lax
from jax.experimental import pallas as pl
from jax.experimental.pallas import tpu as pltpu
from jax.experimental.pallas import tpu_sc as plsc

N = 10000
D = 128
H = 4
C = 32
OUT = H * C
HALF = OUT // 2

NPAD = 10240
NSUB = 16
EB = 128
WREC = HALF + 16
ROWS_PER_SUB = NPAD // NSUB


def _proj_body(x_ref, wl_ref, bl_ref, wr_ref, br_ref,
               xl0_ref, xl1_ref, xr0_ref, xr1_ref):
    xb = x_ref[...]
    xl = jnp.dot(xb, wl_ref[...], preferred_element_type=jnp.float32) + bl_ref[...]
    xr = jnp.dot(xb, wr_ref[...], preferred_element_type=jnp.float32) + br_ref[...]
    xl0_ref[...] = xl[:, :HALF]
    xl1_ref[...] = xl[:, HALF:]
    xr0_ref[...] = xr[:, :HALF]
    xr1_ref[...] = xr[:, HALF:]


def _project(x_pad, Wl, bl, Wr, br):
    blk = 512
    grid = NPAD // blk
    half_spec = pl.BlockSpec((blk, HALF), lambda i: (i, 0))
    half_shape = jax.ShapeDtypeStruct((NPAD, HALF), jnp.float32)
    return pl.pallas_call(
        _proj_body,
        grid=(grid,),
        in_specs=[
            pl.BlockSpec((blk, D), lambda i: (i, 0)),
            pl.BlockSpec((D, OUT), lambda i: (0, 0)),
            pl.BlockSpec((1, OUT), lambda i: (0, 0)),
            pl.BlockSpec((D, OUT), lambda i: (0, 0)),
            pl.BlockSpec((1, OUT), lambda i: (0, 0)),
        ],
        out_specs=[half_spec, half_spec, half_spec, half_spec],
        out_shape=[half_shape, half_shape, half_shape, half_shape],
    )(x_pad, Wl, bl.reshape(1, OUT), Wr, br.reshape(1, OUT))


def _make_edge_kernel(chunks_per_sub):
    mesh = plsc.VectorSubcoreMesh(
        core_axis_name="c", subcore_axis_name="s", num_cores=2, num_subcores=16
    )

    @functools.partial(
        pl.kernel,
        out_type=jax.ShapeDtypeStruct((2, NPAD, WREC), jnp.float32),
        mesh=mesh,
        compiler_params=pltpu.CompilerParams(
            needs_layout_passes=False, use_tc_tiling_on_sc=False
        ),
        scratch_types=[
            pltpu.VMEM((EB,), jnp.int32),
            pltpu.VMEM((EB,), jnp.int32),
            pltpu.VMEM((EB, HALF), jnp.float32),
            pltpu.VMEM((EB, HALF), jnp.float32),
            pltpu.VMEM((EB, WREC), jnp.float32),
            pltpu.VMEM((8, 16), jnp.float32),
            pltpu.VMEM_SHARED((NPAD, WREC), jnp.float32),
            pltpu.SemaphoreType.DMA,
            pltpu.SemaphoreType.DMA,
        ],
    )
    def edge_kernel(
        xl0_hbm, xl1_hbm, xr0_hbm, xr1_hbm, src_hbm, dst_hbm, att_hbm, out_hbm,
        sidx, didx, xlr, xrr, orow, att_vm, table, sem1, sem2,
    ):
        cid = lax.axis_index("c")
        sid = lax.axis_index("s")

        zero16 = jnp.zeros((16,), jnp.float32)

        def zero_row(r, carry):
            for j in range(WREC // 16):
                orow[r, pl.ds(j * 16, 16)] = zero16
            return carry

        lax.fori_loop(0, EB, zero_row, 0)
        for k in range(ROWS_PER_SUB // EB):
            pltpu.sync_copy(orow, table.at[pl.ds(sid * ROWS_PER_SUB + k * EB, EB)])
        plsc.subcore_barrier()

        pltpu.sync_copy(att_hbm, att_vm)
        lane = lax.iota(jnp.int32, 16)

        def sweep(cc, xl_hbm, xr_hbm):
            att_r = [att_vm[4 * cc + k, :] for k in range(4)]

            def chunk_body(j, carry):
                base = (sid * chunks_per_sub + j) * EB
                pltpu.sync_copy(src_hbm.at[pl.ds(base, EB)], sidx)
                pltpu.sync_copy(dst_hbm.at[pl.ds(base, EB)], didx)
                cp1 = pltpu.async_copy(xl_hbm.at[sidx], xlr, sem1)
                cp2 = pltpu.async_copy(xr_hbm.at[didx], xrr, sem2)
                cp1.wait()
                cp2.wait()

                def edge_body(e, ecarry):
                    wrow = zero16
                    for lh in range(2):
                        a0 = xlr[e, pl.ds(lh * 32, 16)]
                        a1 = xlr[e, pl.ds(lh * 32 + 16, 16)]
                        b0 = xrr[e, pl.ds(lh * 32, 16)]
                        b1 = xrr[e, pl.ds(lh * 32 + 16, 16)]
                        t0 = a0 + b0
                        t1 = a1 + b1
                        l0 = jnp.maximum(t0, 0.2 * t0)
                        l1 = jnp.maximum(t1, 0.2 * t1)
                        s = l0 * att_r[2 * lh] + l1 * att_r[2 * lh + 1]
                        sc = jnp.sum(s)
                        wv = jnp.exp(jnp.full((16,), sc, jnp.float32))
                        orow[e, pl.ds(lh * 32, 16)] = a0 * wv
                        orow[e, pl.ds(lh * 32 + 16, 16)] = a1 * wv
                        wrow = jnp.where(lane == lh, wv, wrow)
                    orow[e, pl.ds(HALF, 16)] = wrow
                    return ecarry

                lax.fori_loop(0, EB, edge_body, 0)
                pltpu.sync_copy(orow, table.at[didx], add=True)
                return carry

            lax.fori_loop(0, chunks_per_sub, chunk_body, 0)

        pl.when(cid == 0)(lambda: sweep(0, xl0_hbm, xr0_hbm))
        pl.when(cid == 1)(lambda: sweep(1, xl1_hbm, xr1_hbm))
        plsc.subcore_barrier()

        def copy_out(cc):
            for k in range(ROWS_PER_SUB // EB):
                r0 = sid * ROWS_PER_SUB + k * EB
                pltpu.sync_copy(table.at[pl.ds(r0, EB)], orow)
                pltpu.sync_copy(orow, out_hbm.at[cc, pl.ds(r0, EB)])

        pl.when(cid == 0)(lambda: copy_out(0))
        pl.when(cid == 1)(lambda: copy_out(1))

    return edge_kernel


def _gate_body(
    num_ref, h_ref, exp_ref, bias_ref, w1a_ref, w1b_ref, b1_ref,
    w2a_ref, w2b_ref, b2_ref, out_ref,
):
    n0 = num_ref[0]
    n1 = num_ref[1]
    agg = jnp.concatenate([n0[:, :HALF], n1[:, :HALF]], axis=1)
    den4 = jnp.concatenate([n0[:, HALF:HALF + 2], n1[:, HALF:HALF + 2]], axis=1)
    den = jnp.dot(den4, exp_ref[...], preferred_element_type=jnp.float32)
    f = jax.nn.sigmoid(agg / (den + 1e-16) + bias_ref[...])
    hb = h_ref[...]
    ru = jax.nn.sigmoid(
        jnp.dot(f, w1a_ref[...], preferred_element_type=jnp.float32)
        + jnp.dot(hb, w1b_ref[...], preferred_element_type=jnp.float32)
        + b1_ref[...]
    )
    r = ru[:, :OUT]
    u = ru[:, OUT:]
    cc = jnp.tanh(
        jnp.dot(f, w2a_ref[...], preferred_element_type=jnp.float32)
        + jnp.dot(r * hb, w2b_ref[...], preferred_element_type=jnp.float32)
        + b2_ref[...]
    )
    out_ref[...] = u * hb + (1.0 - u) * cc


def _gate(num2, h_pad, bias, W1, b1, W2, b2):
    blk = 512
    grid = NPAD // blk
    expand = (
        jnp.arange(OUT, dtype=jnp.int32)[None, :] // C
        == jnp.arange(4, dtype=jnp.int32)[:, None]
    ).astype(jnp.float32)
    return pl.pallas_call(
        _gate_body,
        grid=(grid,),
        in_specs=[
            pl.BlockSpec((2, blk, WREC), lambda i: (0, i, 0)),
            pl.BlockSpec((blk, OUT), lambda i: (i, 0)),
            pl.BlockSpec((4, OUT), lambda i: (0, 0)),
            pl.BlockSpec((1, OUT), lambda i: (0, 0)),
            pl.BlockSpec((OUT, 2 * OUT), lambda i: (0, 0)),
            pl.BlockSpec((OUT, 2 * OUT), lambda i: (0, 0)),
            pl.BlockSpec((1, 2 * OUT), lambda i: (0, 0)),
            pl.BlockSpec((OUT, OUT), lambda i: (0, 0)),
            pl.BlockSpec((OUT, OUT), lambda i: (0, 0)),
            pl.BlockSpec((1, OUT), lambda i: (0, 0)),
        ],
        out_specs=pl.BlockSpec((blk, OUT), lambda i: (i, 0)),
        out_shape=jax.ShapeDtypeStruct((NPAD, OUT), jnp.float32),
    )(
        num2, h_pad, expand, bias.reshape(1, OUT),
        W1[:OUT], W1[OUT:], b1.reshape(1, 2 * OUT),
        W2[:OUT], W2[OUT:], b2.reshape(1, OUT),
    )


def kernel(x, edge_index, edge_weight, h, Wl, bl, Wr, br, att, bias, W1, b1, W2, b2):
    del edge_weight

    ne = edge_index.shape[1] + N
    ep = ((ne + NSUB * EB - 1) // (NSUB * EB)) * (NSUB * EB)
    chunks_per_sub = ep // (NSUB * EB)

    loops = jnp.arange(N, dtype=jnp.int32)
    padv = jnp.full((ep - ne,), N, dtype=jnp.int32)
    src = jnp.concatenate([edge_index[0].astype(jnp.int32), loops, padv])
    dst = jnp.concatenate([edge_index[1].astype(jnp.int32), loops, padv])

    x_pad = jnp.pad(x, ((0, NPAD - N), (0, 0)))
    xl0, xl1, xr0, xr1 = _project(x_pad, Wl, bl, Wr, br)

    att2 = att.reshape(8, 16)
    num2 = _make_edge_kernel(chunks_per_sub)(xl0, xl1, xr0, xr1, src, dst, att2)

    h_pad = jnp.pad(h, ((0, NPAD - N), (0, 0)))
    return _gate(num2, h_pad, bias, W1, b1, W2, b2)[:N]


# speedup vs baseline: 83.5429x; 3.2673x over previous
import functools

import jax
import jax.numpy as jnp
from jax import lax
from jax.experimental import pallas as pl
from jax.experimental.pallas import tpu as pltpu
from jax.experimental.pallas import tpu_sc as plsc

N = 10000
D = 128
H = 4
C = 32
OUT = H * C
HALF = OUT // 2

NPAD = 10240
NSUB = 16
EB = 128
WREC = HALF + 16
ROWS_PER_SUB = NPAD // NSUB


def _proj_body(x_ref, wl_ref, bl_ref, wr_ref, br_ref,
               xl0_ref, xl1_ref, xr0_ref, xr1_ref):
    xb = x_ref[...]
    xl = jnp.dot(xb, wl_ref[...], preferred_element_type=jnp.float32) + bl_ref[...]
    xr = jnp.dot(xb, wr_ref[...], preferred_element_type=jnp.float32) + br_ref[...]
    xl0_ref[...] = xl[:, :HALF]
    xl1_ref[...] = xl[:, HALF:]
    xr0_ref[...] = xr[:, :HALF]
    xr1_ref[...] = xr[:, HALF:]


def _project(x_pad, Wl, bl, Wr, br):
    blk = 512
    grid = NPAD // blk
    half_spec = pl.BlockSpec((blk, HALF), lambda i: (i, 0))
    half_shape = jax.ShapeDtypeStruct((NPAD, HALF), jnp.float32)
    return pl.pallas_call(
        _proj_body,
        grid=(grid,),
        in_specs=[
            pl.BlockSpec((blk, D), lambda i: (i, 0)),
            pl.BlockSpec((D, OUT), lambda i: (0, 0)),
            pl.BlockSpec((1, OUT), lambda i: (0, 0)),
            pl.BlockSpec((D, OUT), lambda i: (0, 0)),
            pl.BlockSpec((1, OUT), lambda i: (0, 0)),
        ],
        out_specs=[half_spec, half_spec, half_spec, half_spec],
        out_shape=[half_shape, half_shape, half_shape, half_shape],
    )(x_pad, Wl, bl.reshape(1, OUT), Wr, br.reshape(1, OUT))


def _make_edge_kernel(chunks_per_sub):
    mesh = plsc.VectorSubcoreMesh(
        core_axis_name="c", subcore_axis_name="s", num_cores=2, num_subcores=16
    )

    @functools.partial(
        pl.kernel,
        out_type=jax.ShapeDtypeStruct((2, NPAD, WREC), jnp.float32),
        mesh=mesh,
        compiler_params=pltpu.CompilerParams(
            needs_layout_passes=False, use_tc_tiling_on_sc=False
        ),
        scratch_types=[
            pltpu.VMEM((2, EB), jnp.int32),
            pltpu.VMEM((2, EB), jnp.int32),
            pltpu.VMEM((EB, HALF), jnp.float32),
            pltpu.VMEM((EB, HALF), jnp.float32),
            pltpu.VMEM((EB, HALF), jnp.float32),
            pltpu.VMEM((EB, HALF), jnp.float32),
            pltpu.VMEM((EB, WREC), jnp.float32),
            pltpu.VMEM((8, 16), jnp.float32),
            pltpu.VMEM_SHARED((NPAD, WREC), jnp.float32),
            pltpu.SemaphoreType.DMA,
            pltpu.SemaphoreType.DMA,
        ],
    )
    def edge_kernel(
        xl0_hbm, xl1_hbm, xr0_hbm, xr1_hbm, src_hbm, dst_hbm, att_hbm, out_hbm,
        sidx, didx, xlr0, xlr1, xrr0, xrr1, orow, att_vm, table, sem0, sem1,
    ):
        cid = lax.axis_index("c")
        sid = lax.axis_index("s")

        zero16 = jnp.zeros((16,), jnp.float32)

        def zero_row(r, carry):
            for j in range(WREC // 16):
                orow[r, pl.ds(j * 16, 16)] = zero16
            return carry

        lax.fori_loop(0, EB, zero_row, 0)
        for k in range(ROWS_PER_SUB // EB):
            pltpu.sync_copy(orow, table.at[pl.ds(sid * ROWS_PER_SUB + k * EB, EB)])
        plsc.subcore_barrier()

        pltpu.sync_copy(att_hbm, att_vm)
        lane = lax.iota(jnp.int32, 16)

        def sweep(cc, xl_hbm, xr_hbm):
            att_r = [att_vm[4 * cc + k, :] for k in range(4)]
            gbufs = [(xlr0, xrr0, sem0), (xlr1, xrr1, sem1)]

            def fire(p, j):
                base = (sid * chunks_per_sub + j) * EB
                pltpu.sync_copy(src_hbm.at[pl.ds(base, EB)], sidx.at[p])
                pltpu.sync_copy(dst_hbm.at[pl.ds(base, EB)], didx.at[p])
                xl_, xr_, sem = gbufs[p]
                pltpu.async_copy(xl_hbm.at[sidx.at[p]], xl_, sem)
                pltpu.async_copy(xr_hbm.at[didx.at[p]], xr_, sem)

            def drain(p):
                xl_, xr_, sem = gbufs[p]
                pltpu.make_async_copy(xl_hbm.at[sidx.at[p]], xl_, sem).wait()
                pltpu.make_async_copy(xr_hbm.at[didx.at[p]], xr_, sem).wait()

            def compute(p):
                xl_, xr_, _ = gbufs[p]

                @plsc.parallel_loop(0, EB, unroll=4)
                def edge_body(e):
                    wrow = zero16
                    for lh in range(2):
                        a0 = xl_[e, pl.ds(lh * 32, 16)]
                        a1 = xl_[e, pl.ds(lh * 32 + 16, 16)]
                        b0 = xr_[e, pl.ds(lh * 32, 16)]
                        b1 = xr_[e, pl.ds(lh * 32 + 16, 16)]
                        t0 = a0 + b0
                        t1 = a1 + b1
                        l0 = jnp.maximum(t0, 0.2 * t0)
                        l1 = jnp.maximum(t1, 0.2 * t1)
                        s = l0 * att_r[2 * lh] + l1 * att_r[2 * lh + 1]
                        sc = jnp.sum(s)
                        wv = jnp.exp(jnp.full((16,), sc, jnp.float32))
                        orow[e, pl.ds(lh * 32, 16)] = a0 * wv
                        orow[e, pl.ds(lh * 32 + 16, 16)] = a1 * wv
                        wrow = jnp.where(lane == lh, wv, wrow)
                    orow[e, pl.ds(HALF, 16)] = wrow

                pltpu.sync_copy(orow, table.at[didx.at[p]], add=True)

            fire(0, 0)

            def pair_body(jj, carry):
                j = 2 * jj
                fire(1, j + 1)
                drain(0)
                compute(0)

                @pl.when(j + 2 < chunks_per_sub)
                def _():
                    fire(0, j + 2)

                drain(1)
                compute(1)
                return carry

            lax.fori_loop(0, chunks_per_sub // 2, pair_body, 0)

        pl.when(cid == 0)(lambda: sweep(0, xl0_hbm, xr0_hbm))
        pl.when(cid == 1)(lambda: sweep(1, xl1_hbm, xr1_hbm))
        plsc.subcore_barrier()

        def copy_out(cc):
            for k in range(ROWS_PER_SUB // EB):
                r0 = sid * ROWS_PER_SUB + k * EB
                pltpu.sync_copy(table.at[pl.ds(r0, EB)], orow)
                pltpu.sync_copy(orow, out_hbm.at[cc, pl.ds(r0, EB)])

        pl.when(cid == 0)(lambda: copy_out(0))
        pl.when(cid == 1)(lambda: copy_out(1))

    return edge_kernel


def _gate_body(
    num_ref, h_ref, exp_ref, bias_ref, w1a_ref, w1b_ref, b1_ref,
    w2a_ref, w2b_ref, b2_ref, out_ref,
):
    n0 = num_ref[0]
    n1 = num_ref[1]
    agg = jnp.concatenate([n0[:, :HALF], n1[:, :HALF]], axis=1)
    den4 = jnp.concatenate([n0[:, HALF:HALF + 2], n1[:, HALF:HALF + 2]], axis=1)
    den = jnp.dot(den4, exp_ref[...], preferred_element_type=jnp.float32)
    f = jax.nn.sigmoid(agg / (den + 1e-16) + bias_ref[...])
    hb = h_ref[...]
    ru = jax.nn.sigmoid(
        jnp.dot(f, w1a_ref[...], preferred_element_type=jnp.float32)
        + jnp.dot(hb, w1b_ref[...], preferred_element_type=jnp.float32)
        + b1_ref[...]
    )
    r = ru[:, :OUT]
    u = ru[:, OUT:]
    cc = jnp.tanh(
        jnp.dot(f, w2a_ref[...], preferred_element_type=jnp.float32)
        + jnp.dot(r * hb, w2b_ref[...], preferred_element_type=jnp.float32)
        + b2_ref[...]
    )
    out_ref[...] = u * hb + (1.0 - u) * cc


def _gate(num2, h_pad, bias, W1, b1, W2, b2):
    blk = 512
    grid = NPAD // blk
    expand = (
        jnp.arange(OUT, dtype=jnp.int32)[None, :] // C
        == jnp.arange(4, dtype=jnp.int32)[:, None]
    ).astype(jnp.float32)
    return pl.pallas_call(
        _gate_body,
        grid=(grid,),
        in_specs=[
            pl.BlockSpec((2, blk, WREC), lambda i: (0, i, 0)),
            pl.BlockSpec((blk, OUT), lambda i: (i, 0)),
            pl.BlockSpec((4, OUT), lambda i: (0, 0)),
            pl.BlockSpec((1, OUT), lambda i: (0, 0)),
            pl.BlockSpec((OUT, 2 * OUT), lambda i: (0, 0)),
            pl.BlockSpec((OUT, 2 * OUT), lambda i: (0, 0)),
            pl.BlockSpec((1, 2 * OUT), lambda i: (0, 0)),
            pl.BlockSpec((OUT, OUT), lambda i: (0, 0)),
            pl.BlockSpec((OUT, OUT), lambda i: (0, 0)),
            pl.BlockSpec((1, OUT), lambda i: (0, 0)),
        ],
        out_specs=pl.BlockSpec((blk, OUT), lambda i: (i, 0)),
        out_shape=jax.ShapeDtypeStruct((NPAD, OUT), jnp.float32),
    )(
        num2, h_pad, expand, bias.reshape(1, OUT),
        W1[:OUT], W1[OUT:], b1.reshape(1, 2 * OUT),
        W2[:OUT], W2[OUT:], b2.reshape(1, OUT),
    )


def kernel(x, edge_index, edge_weight, h, Wl, bl, Wr, br, att, bias, W1, b1, W2, b2):
    del edge_weight

    ne = edge_index.shape[1] + N
    q = 2 * NSUB * EB
    ep = ((ne + q - 1) // q) * q
    chunks_per_sub = ep // (NSUB * EB)

    loops = jnp.arange(N, dtype=jnp.int32)
    padv = jnp.full((ep - ne,), N, dtype=jnp.int32)
    src = jnp.concatenate([edge_index[0].astype(jnp.int32), loops, padv])
    dst = jnp.concatenate([edge_index[1].astype(jnp.int32), loops, padv])

    x_pad = jnp.pad(x, ((0, NPAD - N), (0, 0)))
    xl0, xl1, xr0, xr1 = _project(x_pad, Wl, bl, Wr, br)

    att2 = att.reshape(8, 16)
    num2 = _make_edge_kernel(chunks_per_sub)(xl0, xl1, xr0, xr1, src, dst, att2)

    h_pad = jnp.pad(h, ((0, NPAD - N), (0, 0)))
    return _gate(num2, h_pad, bias, W1, b1, W2, b2)[:N]


# speedup vs baseline: 112.2217x; 1.3433x over previous
import functools

import jax
import jax.numpy as jnp
from jax import lax
from jax.experimental import pallas as pl
from jax.experimental.pallas import tpu as pltpu
from jax.experimental.pallas import tpu_sc as plsc

N = 10000
D = 128
H = 4
C = 32
OUT = H * C
HALF = OUT // 2

NPAD = 10240
NSUB = 16
EB = 128
WREC = HALF + 16
ROWS_PER_SUB = NPAD // NSUB


def _proj_body(x_ref, wl_ref, bl_ref, wr_ref, br_ref,
               xl0_ref, xl1_ref, xr0_ref, xr1_ref):
    xb = x_ref[...]
    xl = jnp.dot(xb, wl_ref[...], preferred_element_type=jnp.float32) + bl_ref[...]
    xr = jnp.dot(xb, wr_ref[...], preferred_element_type=jnp.float32) + br_ref[...]
    xl0_ref[...] = xl[:, :HALF]
    xl1_ref[...] = xl[:, HALF:]
    xr0_ref[...] = xr[:, :HALF]
    xr1_ref[...] = xr[:, HALF:]


def _project(x_pad, Wl, bl, Wr, br):
    blk = 512
    grid = NPAD // blk
    half_spec = pl.BlockSpec((blk, HALF), lambda i: (i, 0))
    half_shape = jax.ShapeDtypeStruct((NPAD, HALF), jnp.float32)
    return pl.pallas_call(
        _proj_body,
        grid=(grid,),
        in_specs=[
            pl.BlockSpec((blk, D), lambda i: (i, 0)),
            pl.BlockSpec((D, OUT), lambda i: (0, 0)),
            pl.BlockSpec((1, OUT), lambda i: (0, 0)),
            pl.BlockSpec((D, OUT), lambda i: (0, 0)),
            pl.BlockSpec((1, OUT), lambda i: (0, 0)),
        ],
        out_specs=[half_spec, half_spec, half_spec, half_spec],
        out_shape=[half_shape, half_shape, half_shape, half_shape],
    )(x_pad, Wl, bl.reshape(1, OUT), Wr, br.reshape(1, OUT))


def _make_edge_kernel(chunks_per_sub):
    mesh = plsc.VectorSubcoreMesh(
        core_axis_name="c", subcore_axis_name="s", num_cores=2, num_subcores=16
    )

    @functools.partial(
        pl.kernel,
        out_type=jax.ShapeDtypeStruct((2, NPAD, WREC), jnp.float32),
        mesh=mesh,
        compiler_params=pltpu.CompilerParams(
            needs_layout_passes=False, use_tc_tiling_on_sc=False
        ),
        scratch_types=[
            pltpu.VMEM((chunks_per_sub // 3, EB), jnp.int32),
            pltpu.VMEM((chunks_per_sub // 3, EB), jnp.int32),
            pltpu.VMEM((EB, HALF), jnp.float32),
            pltpu.VMEM((EB, HALF), jnp.float32),
            pltpu.VMEM((EB, HALF), jnp.float32),
            pltpu.VMEM((EB, HALF), jnp.float32),
            pltpu.VMEM((EB, WREC), jnp.float32),
            pltpu.VMEM((EB, WREC), jnp.float32),
            pltpu.VMEM((8, 16), jnp.float32),
            pltpu.VMEM_SHARED((NPAD, WREC), jnp.float32),
            pltpu.SemaphoreType.DMA,
            pltpu.SemaphoreType.DMA,
            pltpu.SemaphoreType.DMA,
            pltpu.SemaphoreType.DMA,
        ],
    )
    def edge_kernel(
        xl0_hbm, xl1_hbm, xr0_hbm, xr1_hbm, src_hbm, dst_hbm, att_hbm, out_hbm,
        sidx, didx, xlr0, xlr1, xrr0, xrr1, orow0, orow1, att_vm, table,
        sem0, sem1, scsem0, scsem1,
    ):
        cid = lax.axis_index("c")
        sid = lax.axis_index("s")

        zero16 = jnp.zeros((16,), jnp.float32)

        def zero_row(r, carry):
            for j in range(WREC // 16):
                orow0[r, pl.ds(j * 16, 16)] = zero16
            return carry

        lax.fori_loop(0, EB, zero_row, 0)
        for k in range(ROWS_PER_SUB // EB):
            pltpu.sync_copy(orow0, table.at[pl.ds(sid * ROWS_PER_SUB + k * EB, EB)])
        plsc.subcore_barrier()

        pltpu.sync_copy(att_hbm, att_vm)
        lane = lax.iota(jnp.int32, 16)

        def sweep(cc, xl_hbm, xr_hbm):
            att_r = [att_vm[4 * cc + k, :] for k in range(4)]
            gbufs = [(xlr0, xrr0, sem0), (xlr1, xrr1, sem1)]
            obufs = [(orow0, scsem0), (orow1, scsem1)]
            ssc = chunks_per_sub // 3

            def fire(p, j):
                xl_, xr_, sem = gbufs[p]
                pltpu.async_copy(xl_hbm.at[sidx.at[j]], xl_, sem)
                pltpu.async_copy(xr_hbm.at[didx.at[j]], xr_, sem)

            def drain_gather(p, j):
                xl_, xr_, sem = gbufs[p]
                pltpu.make_async_copy(xl_hbm.at[sidx.at[j]], xl_, sem).wait()
                pltpu.make_async_copy(xr_hbm.at[didx.at[j]], xr_, sem).wait()

            def compute(p, j, guard):
                xl_, xr_, _ = gbufs[p]
                o_, ssem = obufs[p]

                @pl.when(guard)
                def _drain_prev_scatter():
                    pltpu.make_async_copy(o_, table.at[didx.at[j]], ssem).wait()

                @plsc.parallel_loop(0, EB, unroll=8)
                def edge_body(e):
                    wrow = zero16
                    for lh in range(2):
                        a0 = xl_[e, pl.ds(lh * 32, 16)]
                        a1 = xl_[e, pl.ds(lh * 32 + 16, 16)]
                        b0 = xr_[e, pl.ds(lh * 32, 16)]
                        b1 = xr_[e, pl.ds(lh * 32 + 16, 16)]
                        t0 = a0 + b0
                        t1 = a1 + b1
                        l0 = jnp.maximum(t0, 0.2 * t0)
                        l1 = jnp.maximum(t1, 0.2 * t1)
                        s = l0 * att_r[2 * lh] + l1 * att_r[2 * lh + 1]
                        sc = jnp.sum(s)
                        wv = jnp.exp(jnp.full((16,), sc, jnp.float32))
                        o_[e, pl.ds(lh * 32, 16)] = a0 * wv
                        o_[e, pl.ds(lh * 32 + 16, 16)] = a1 * wv
                        wrow = jnp.where(lane == lh, wv, wrow)
                    o_[e, pl.ds(HALF, 16)] = wrow

                pltpu.async_copy(o_, table.at[didx.at[j]], ssem, add=True)

            def pair_body(jj, carry):
                j = 2 * jj
                fire(1, j + 1)
                drain_gather(0, j)
                compute(0, j, jj > 0)

                @pl.when(j + 2 < ssc)
                def _():
                    fire(0, j + 2)

                drain_gather(1, j + 1)
                compute(1, j + 1, jj > 0)
                return carry

            for sci in range(3):
                row0 = sid * chunks_per_sub + sci * ssc
                pltpu.sync_copy(src_hbm.at[pl.ds(row0, ssc)], sidx)
                pltpu.sync_copy(dst_hbm.at[pl.ds(row0, ssc)], didx)
                fire(0, 0)
                lax.fori_loop(0, ssc // 2, pair_body, 0)
                pltpu.make_async_copy(orow0, table.at[didx.at[0]], scsem0).wait()
                pltpu.make_async_copy(orow1, table.at[didx.at[0]], scsem1).wait()

        pl.when(cid == 0)(lambda: sweep(0, xl0_hbm, xr0_hbm))
        pl.when(cid == 1)(lambda: sweep(1, xl1_hbm, xr1_hbm))
        plsc.subcore_barrier()

        def copy_out(cc):
            for k in range(ROWS_PER_SUB // EB):
                r0 = sid * ROWS_PER_SUB + k * EB
                pltpu.sync_copy(table.at[pl.ds(r0, EB)], orow0)
                pltpu.sync_copy(orow0, out_hbm.at[cc, pl.ds(r0, EB)])

        pl.when(cid == 0)(lambda: copy_out(0))
        pl.when(cid == 1)(lambda: copy_out(1))

    return edge_kernel


def _gate_body(
    num_ref, h_ref, exp_ref, bias_ref, w1a_ref, w1b_ref, b1_ref,
    w2a_ref, w2b_ref, b2_ref, out_ref,
):
    n0 = num_ref[0]
    n1 = num_ref[1]
    agg = jnp.concatenate([n0[:, :HALF], n1[:, :HALF]], axis=1)
    den4 = jnp.concatenate([n0[:, HALF:HALF + 2], n1[:, HALF:HALF + 2]], axis=1)
    den = jnp.dot(den4, exp_ref[...], preferred_element_type=jnp.float32)
    f = jax.nn.sigmoid(agg / (den + 1e-16) + bias_ref[...])
    hb = h_ref[...]
    ru = jax.nn.sigmoid(
        jnp.dot(f, w1a_ref[...], preferred_element_type=jnp.float32)
        + jnp.dot(hb, w1b_ref[...], preferred_element_type=jnp.float32)
        + b1_ref[...]
    )
    r = ru[:, :OUT]
    u = ru[:, OUT:]
    cc = jnp.tanh(
        jnp.dot(f, w2a_ref[...], preferred_element_type=jnp.float32)
        + jnp.dot(r * hb, w2b_ref[...], preferred_element_type=jnp.float32)
        + b2_ref[...]
    )
    out_ref[...] = u * hb + (1.0 - u) * cc


def _gate(num2, h_pad, bias, W1, b1, W2, b2):
    blk = 512
    grid = NPAD // blk
    expand = (
        jnp.arange(OUT, dtype=jnp.int32)[None, :] // C
        == jnp.arange(4, dtype=jnp.int32)[:, None]
    ).astype(jnp.float32)
    return pl.pallas_call(
        _gate_body,
        grid=(grid,),
        in_specs=[
            pl.BlockSpec((2, blk, WREC), lambda i: (0, i, 0)),
            pl.BlockSpec((blk, OUT), lambda i: (i, 0)),
            pl.BlockSpec((4, OUT), lambda i: (0, 0)),
            pl.BlockSpec((1, OUT), lambda i: (0, 0)),
            pl.BlockSpec((OUT, 2 * OUT), lambda i: (0, 0)),
            pl.BlockSpec((OUT, 2 * OUT), lambda i: (0, 0)),
            pl.BlockSpec((1, 2 * OUT), lambda i: (0, 0)),
            pl.BlockSpec((OUT, OUT), lambda i: (0, 0)),
            pl.BlockSpec((OUT, OUT), lambda i: (0, 0)),
            pl.BlockSpec((1, OUT), lambda i: (0, 0)),
        ],
        out_specs=pl.BlockSpec((blk, OUT), lambda i: (i, 0)),
        out_shape=jax.ShapeDtypeStruct((NPAD, OUT), jnp.float32),
    )(
        num2, h_pad, expand, bias.reshape(1, OUT),
        W1[:OUT], W1[OUT:], b1.reshape(1, 2 * OUT),
        W2[:OUT], W2[OUT:], b2.reshape(1, OUT),
    )


def kernel(x, edge_index, edge_weight, h, Wl, bl, Wr, br, att, bias, W1, b1, W2, b2):
    del edge_weight

    ne = edge_index.shape[1] + N
    q = 6 * NSUB * EB
    ep = ((ne + q - 1) // q) * q
    chunks_per_sub = ep // (NSUB * EB)

    loops = jnp.arange(N, dtype=jnp.int32)
    padv = jnp.full((ep - ne,), N, dtype=jnp.int32)
    src = jnp.concatenate(
        [edge_index[0].astype(jnp.int32), loops, padv]
    ).reshape(-1, EB)
    dst = jnp.concatenate(
        [edge_index[1].astype(jnp.int32), loops, padv]
    ).reshape(-1, EB)

    x_pad = jnp.pad(x, ((0, NPAD - N), (0, 0)))
    xl0, xl1, xr0, xr1 = _project(x_pad, Wl, bl, Wr, br)

    att2 = att.reshape(8, 16)
    num2 = _make_edge_kernel(chunks_per_sub)(xl0, xl1, xr0, xr1, src, dst, att2)

    h_pad = jnp.pad(h, ((0, NPAD - N), (0, 0)))
    return _gate(num2, h_pad, bias, W1, b1, W2, b2)[:N]


# speedup vs baseline: 120.3728x; 1.0726x over previous
import functools

import jax
import jax.numpy as jnp
from jax import lax
from jax.experimental import pallas as pl
from jax.experimental.pallas import tpu as pltpu
from jax.experimental.pallas import tpu_sc as plsc

N = 10000
D = 128
H = 4
C = 32
OUT = H * C
HALF = OUT // 2

NPAD = 10240
NSUB = 16
EB = 128
WREC = HALF + 16
ROWS_PER_SUB = NPAD // NSUB


def _proj_body(x_ref, wl_ref, bl_ref, wr_ref, br_ref,
               xl0_ref, xl1_ref, xr0_ref, xr1_ref):
    xb = x_ref[...]
    xl = jnp.dot(xb, wl_ref[...], preferred_element_type=jnp.float32) + bl_ref[...]
    xr = jnp.dot(xb, wr_ref[...], preferred_element_type=jnp.float32) + br_ref[...]
    xl0_ref[...] = xl[:, :HALF]
    xl1_ref[...] = xl[:, HALF:]
    xr0_ref[...] = xr[:, :HALF]
    xr1_ref[...] = xr[:, HALF:]


def _project(x, Wl, bl, Wr, br):
    blk = 1000
    grid = N // blk
    half_spec = pl.BlockSpec((blk, HALF), lambda i: (i, 0))
    half_shape = jax.ShapeDtypeStruct((N + 8, HALF), jnp.float32)
    return pl.pallas_call(
        _proj_body,
        grid=(grid,),
        in_specs=[
            pl.BlockSpec((blk, D), lambda i: (i, 0)),
            pl.BlockSpec((D, OUT), lambda i: (0, 0)),
            pl.BlockSpec((1, OUT), lambda i: (0, 0)),
            pl.BlockSpec((D, OUT), lambda i: (0, 0)),
            pl.BlockSpec((1, OUT), lambda i: (0, 0)),
        ],
        out_specs=[half_spec, half_spec, half_spec, half_spec],
        out_shape=[half_shape, half_shape, half_shape, half_shape],
    )(x, Wl, bl.reshape(1, OUT), Wr, br.reshape(1, OUT))


def _make_edge_kernel(chunks_per_sub):
    mesh = plsc.VectorSubcoreMesh(
        core_axis_name="c", subcore_axis_name="s", num_cores=2, num_subcores=16
    )

    @functools.partial(
        pl.kernel,
        out_type=jax.ShapeDtypeStruct((2, NPAD, WREC), jnp.float32),
        mesh=mesh,
        compiler_params=pltpu.CompilerParams(
            needs_layout_passes=False, use_tc_tiling_on_sc=False
        ),
        scratch_types=[
            pltpu.VMEM((chunks_per_sub // 3, EB), jnp.int32),
            pltpu.VMEM((chunks_per_sub // 3, EB), jnp.int32),
            pltpu.VMEM((EB, HALF), jnp.float32),
            pltpu.VMEM((EB, HALF), jnp.float32),
            pltpu.VMEM((EB, HALF), jnp.float32),
            pltpu.VMEM((EB, HALF), jnp.float32),
            pltpu.VMEM((EB, WREC), jnp.float32),
            pltpu.VMEM((EB, WREC), jnp.float32),
            pltpu.VMEM((8, 16), jnp.float32),
            pltpu.VMEM_SHARED((NPAD, WREC), jnp.float32),
            pltpu.SemaphoreType.DMA,
            pltpu.SemaphoreType.DMA,
            pltpu.SemaphoreType.DMA,
            pltpu.SemaphoreType.DMA,
        ],
    )
    def edge_kernel(
        xl0_hbm, xl1_hbm, xr0_hbm, xr1_hbm, src_hbm, dst_hbm, att_hbm, out_hbm,
        sidx, didx, xlr0, xlr1, xrr0, xrr1, orow0, orow1, att_vm, table,
        sem0, sem1, scsem0, scsem1,
    ):
        cid = lax.axis_index("c")
        sid = lax.axis_index("s")

        zero16 = jnp.zeros((16,), jnp.float32)

        def zero_row(r, carry):
            for j in range(WREC // 16):
                orow0[r, pl.ds(j * 16, 16)] = zero16
            return carry

        lax.fori_loop(0, EB, zero_row, 0)
        for k in range(ROWS_PER_SUB // EB):
            pltpu.sync_copy(orow0, table.at[pl.ds(sid * ROWS_PER_SUB + k * EB, EB)])
        plsc.subcore_barrier()

        pltpu.sync_copy(att_hbm, att_vm)
        lane = lax.iota(jnp.int32, 16)

        def sweep(cc, xl_hbm, xr_hbm):
            att_r = [att_vm[4 * cc + k, :] for k in range(4)]
            gbufs = [(xlr0, xrr0, sem0), (xlr1, xrr1, sem1)]
            obufs = [(orow0, scsem0), (orow1, scsem1)]
            ssc = chunks_per_sub // 3

            def fire(p, j):
                xl_, xr_, sem = gbufs[p]
                pltpu.async_copy(xl_hbm.at[sidx.at[j]], xl_, sem)
                pltpu.async_copy(xr_hbm.at[didx.at[j]], xr_, sem)

            def drain_gather(p, j):
                xl_, xr_, sem = gbufs[p]
                pltpu.make_async_copy(xl_hbm.at[sidx.at[j]], xl_, sem).wait()
                pltpu.make_async_copy(xr_hbm.at[didx.at[j]], xr_, sem).wait()

            def compute(p, j, guard):
                xl_, xr_, _ = gbufs[p]
                o_, ssem = obufs[p]

                @pl.when(guard)
                def _drain_prev_scatter():
                    pltpu.make_async_copy(o_, table.at[didx.at[j]], ssem).wait()

                @plsc.parallel_loop(0, EB, unroll=8)
                def edge_body(e):
                    wrow = zero16
                    for lh in range(2):
                        a0 = xl_[e, pl.ds(lh * 32, 16)]
                        a1 = xl_[e, pl.ds(lh * 32 + 16, 16)]
                        b0 = xr_[e, pl.ds(lh * 32, 16)]
                        b1 = xr_[e, pl.ds(lh * 32 + 16, 16)]
                        t0 = a0 + b0
                        t1 = a1 + b1
                        l0 = jnp.maximum(t0, 0.2 * t0)
                        l1 = jnp.maximum(t1, 0.2 * t1)
                        s = l0 * att_r[2 * lh] + l1 * att_r[2 * lh + 1]
                        sc = jnp.sum(s)
                        wv = jnp.exp(jnp.full((16,), sc, jnp.float32))
                        o_[e, pl.ds(lh * 32, 16)] = a0 * wv
                        o_[e, pl.ds(lh * 32 + 16, 16)] = a1 * wv
                        wrow = jnp.where(lane == lh, wv, wrow)
                    o_[e, pl.ds(HALF, 16)] = wrow

                pltpu.async_copy(o_, table.at[didx.at[j]], ssem, add=True)

            def pair_body(jj, carry):
                j = 2 * jj
                fire(1, j + 1)
                drain_gather(0, j)
                compute(0, j, jj > 0)

                @pl.when(j + 2 < ssc)
                def _():
                    fire(0, j + 2)

                drain_gather(1, j + 1)
                compute(1, j + 1, jj > 0)
                return carry

            for sci in range(3):
                row0 = sid * chunks_per_sub + sci * ssc
                pltpu.sync_copy(src_hbm.at[pl.ds(row0, ssc)], sidx)
                pltpu.sync_copy(dst_hbm.at[pl.ds(row0, ssc)], didx)
                fire(0, 0)
                lax.fori_loop(0, ssc // 2, pair_body, 0)
                pltpu.make_async_copy(orow0, table.at[didx.at[0]], scsem0).wait()
                pltpu.make_async_copy(orow1, table.at[didx.at[0]], scsem1).wait()

        pl.when(cid == 0)(lambda: sweep(0, xl0_hbm, xr0_hbm))
        pl.when(cid == 1)(lambda: sweep(1, xl1_hbm, xr1_hbm))
        plsc.subcore_barrier()

        def copy_out(cc):
            for k in range(ROWS_PER_SUB // EB):
                r0 = sid * ROWS_PER_SUB + k * EB
                pltpu.sync_copy(table.at[pl.ds(r0, EB)], orow0)
                pltpu.sync_copy(orow0, out_hbm.at[cc, pl.ds(r0, EB)])

        pl.when(cid == 0)(lambda: copy_out(0))
        pl.when(cid == 1)(lambda: copy_out(1))

    return edge_kernel


def _gate_body(
    num_ref, h_ref, exp_ref, bias_ref, w1a_ref, w1b_ref, b1_ref,
    w2a_ref, w2b_ref, b2_ref, out_ref,
):
    n0 = num_ref[0]
    n1 = num_ref[1]
    agg = jnp.concatenate([n0[:, :HALF], n1[:, :HALF]], axis=1)
    den4 = jnp.concatenate([n0[:, HALF:HALF + 2], n1[:, HALF:HALF + 2]], axis=1)
    den = jnp.dot(den4, exp_ref[...], preferred_element_type=jnp.float32)
    f = jax.nn.sigmoid(agg / (den + 1e-16) + bias_ref[...])
    hb = h_ref[...]
    ru = jax.nn.sigmoid(
        jnp.dot(f, w1a_ref[...], preferred_element_type=jnp.float32)
        + jnp.dot(hb, w1b_ref[...], preferred_element_type=jnp.float32)
        + b1_ref[...]
    )
    r = ru[:, :OUT]
    u = ru[:, OUT:]
    cc = jnp.tanh(
        jnp.dot(f, w2a_ref[...], preferred_element_type=jnp.float32)
        + jnp.dot(r * hb, w2b_ref[...], preferred_element_type=jnp.float32)
        + b2_ref[...]
    )
    out_ref[...] = u * hb + (1.0 - u) * cc


def _gate(num2, h, bias, W1, b1, W2, b2):
    blk = 1000
    grid = N // blk
    expand = (
        jnp.arange(OUT, dtype=jnp.int32)[None, :] // C
        == jnp.arange(4, dtype=jnp.int32)[:, None]
    ).astype(jnp.float32)
    return pl.pallas_call(
        _gate_body,
        grid=(grid,),
        in_specs=[
            pl.BlockSpec((2, blk, WREC), lambda i: (0, i, 0)),
            pl.BlockSpec((blk, OUT), lambda i: (i, 0)),
            pl.BlockSpec((4, OUT), lambda i: (0, 0)),
            pl.BlockSpec((1, OUT), lambda i: (0, 0)),
            pl.BlockSpec((OUT, 2 * OUT), lambda i: (0, 0)),
            pl.BlockSpec((OUT, 2 * OUT), lambda i: (0, 0)),
            pl.BlockSpec((1, 2 * OUT), lambda i: (0, 0)),
            pl.BlockSpec((OUT, OUT), lambda i: (0, 0)),
            pl.BlockSpec((OUT, OUT), lambda i: (0, 0)),
            pl.BlockSpec((1, OUT), lambda i: (0, 0)),
        ],
        out_specs=pl.BlockSpec((blk, OUT), lambda i: (i, 0)),
        out_shape=jax.ShapeDtypeStruct((N, OUT), jnp.float32),
    )(
        num2, h, expand, bias.reshape(1, OUT),
        W1[:OUT], W1[OUT:], b1.reshape(1, 2 * OUT),
        W2[:OUT], W2[OUT:], b2.reshape(1, OUT),
    )


def kernel(x, edge_index, edge_weight, h, Wl, bl, Wr, br, att, bias, W1, b1, W2, b2):
    del edge_weight

    ne = edge_index.shape[1] + N
    q = 6 * NSUB * EB
    ep = ((ne + q - 1) // q) * q
    chunks_per_sub = ep // (NSUB * EB)

    loops = jnp.arange(N, dtype=jnp.int32)
    padv = jnp.full((ep - ne,), N, dtype=jnp.int32)
    src = jnp.concatenate(
        [edge_index[0].astype(jnp.int32), loops, padv]
    ).reshape(-1, EB)
    dst = jnp.concatenate(
        [edge_index[1].astype(jnp.int32), loops, padv]
    ).reshape(-1, EB)

    xl0, xl1, xr0, xr1 = _project(x, Wl, bl, Wr, br)

    att2 = att.reshape(8, 16)
    num2 = _make_edge_kernel(chunks_per_sub)(xl0, xl1, xr0, xr1, src, dst, att2)

    return _gate(num2, h, bias, W1, b1, W2, b2)


# speedup vs baseline: 135.0293x; 1.1218x over previous
import functools

import jax
import jax.numpy as jnp
from jax import lax
from jax.experimental import pallas as pl
from jax.experimental.pallas import tpu as pltpu
from jax.experimental.pallas import tpu_sc as plsc

N = 10000
D = 128
H = 4
C = 32
OUT = H * C
HALF = OUT // 2

NPAD = 10240
NSUB = 16
EB = 128
WREC = HALF + 16
ROWS_PER_SUB = NPAD // NSUB


def _proj_body(x_ref, wl_ref, bl_ref, wr_ref, br_ref,
               xl0_ref, xl1_ref, xr0_ref, xr1_ref):
    xb = x_ref[...]
    xl = jnp.dot(xb, wl_ref[...], preferred_element_type=jnp.float32) + bl_ref[...]
    xr = jnp.dot(xb, wr_ref[...], preferred_element_type=jnp.float32) + br_ref[...]
    xl0_ref[...] = xl[:, :HALF].astype(jnp.bfloat16)
    xl1_ref[...] = xl[:, HALF:].astype(jnp.bfloat16)
    xr0_ref[...] = xr[:, :HALF].astype(jnp.bfloat16)
    xr1_ref[...] = xr[:, HALF:].astype(jnp.bfloat16)


def _project(x, Wl, bl, Wr, br):
    blk = 1000
    grid = N // blk
    half_spec = pl.BlockSpec((blk, HALF), lambda i: (i, 0))
    half_shape = jax.ShapeDtypeStruct((N + 8, HALF), jnp.bfloat16)
    return pl.pallas_call(
        _proj_body,
        grid=(grid,),
        in_specs=[
            pl.BlockSpec((blk, D), lambda i: (i, 0)),
            pl.BlockSpec((D, OUT), lambda i: (0, 0)),
            pl.BlockSpec((1, OUT), lambda i: (0, 0)),
            pl.BlockSpec((D, OUT), lambda i: (0, 0)),
            pl.BlockSpec((1, OUT), lambda i: (0, 0)),
        ],
        out_specs=[half_spec, half_spec, half_spec, half_spec],
        out_shape=[half_shape, half_shape, half_shape, half_shape],
    )(x, Wl, bl.reshape(1, OUT), Wr, br.reshape(1, OUT))


def _make_edge_kernel(chunks_per_sub):
    mesh = plsc.VectorSubcoreMesh(
        core_axis_name="c", subcore_axis_name="s", num_cores=2, num_subcores=16
    )

    @functools.partial(
        pl.kernel,
        out_type=jax.ShapeDtypeStruct((2, NPAD, WREC), jnp.float32),
        mesh=mesh,
        compiler_params=pltpu.CompilerParams(
            needs_layout_passes=False, use_tc_tiling_on_sc=False
        ),
        scratch_types=[
            pltpu.VMEM((chunks_per_sub // 3, EB), jnp.int32),
            pltpu.VMEM((chunks_per_sub // 3, EB), jnp.int32),
            pltpu.VMEM((EB, HALF), jnp.bfloat16),
            pltpu.VMEM((EB, HALF), jnp.bfloat16),
            pltpu.VMEM((EB, HALF), jnp.bfloat16),
            pltpu.VMEM((EB, HALF), jnp.bfloat16),
            pltpu.VMEM((EB, WREC), jnp.float32),
            pltpu.VMEM((EB, WREC), jnp.float32),
            pltpu.VMEM((8, 16), jnp.float32),
            pltpu.VMEM_SHARED((NPAD, WREC), jnp.float32),
            pltpu.SemaphoreType.DMA,
            pltpu.SemaphoreType.DMA,
            pltpu.SemaphoreType.DMA,
            pltpu.SemaphoreType.DMA,
        ],
    )
    def edge_kernel(
        xl0_hbm, xl1_hbm, xr0_hbm, xr1_hbm, src_hbm, dst_hbm, att_hbm, out_hbm,
        sidx, didx, xlr0, xlr1, xrr0, xrr1, orow0, orow1, att_vm, table,
        sem0, sem1, scsem0, scsem1,
    ):
        cid = lax.axis_index("c")
        sid = lax.axis_index("s")

        zero16 = jnp.zeros((16,), jnp.float32)

        def zero_row(r, carry):
            for j in range(WREC // 16):
                orow0[r, pl.ds(j * 16, 16)] = zero16
            return carry

        lax.fori_loop(0, EB, zero_row, 0)
        for k in range(ROWS_PER_SUB // EB):
            pltpu.sync_copy(orow0, table.at[pl.ds(sid * ROWS_PER_SUB + k * EB, EB)])
        plsc.subcore_barrier()

        pltpu.sync_copy(att_hbm, att_vm)
        lane = lax.iota(jnp.int32, 16)

        def sweep(cc, xl_hbm, xr_hbm):
            att_r = [att_vm[4 * cc + k, :] for k in range(4)]
            gbufs = [(xlr0, xrr0, sem0), (xlr1, xrr1, sem1)]
            obufs = [(orow0, scsem0), (orow1, scsem1)]
            ssc = chunks_per_sub // 3

            def fire(p, j):
                xl_, xr_, sem = gbufs[p]
                pltpu.async_copy(xl_hbm.at[sidx.at[j]], xl_, sem)
                pltpu.async_copy(xr_hbm.at[didx.at[j]], xr_, sem)

            def drain_gather(p, j):
                xl_, xr_, sem = gbufs[p]
                pltpu.make_async_copy(xl_hbm.at[sidx.at[j]], xl_, sem).wait()
                pltpu.make_async_copy(xr_hbm.at[didx.at[j]], xr_, sem).wait()

            def compute(p, j, guard):
                xl_, xr_, _ = gbufs[p]
                o_, ssem = obufs[p]

                @pl.when(guard)
                def _drain_prev_scatter():
                    pltpu.make_async_copy(o_, table.at[didx.at[j]], ssem).wait()

                @plsc.parallel_loop(0, EB, unroll=8)
                def edge_body(e):
                    wrow = zero16
                    for lh in range(2):
                        a0, a1 = plsc.unpack(
                            xl_[e, pl.ds(lh * 32, 32)],
                            format=plsc.PackFormat.INTERLEAVED,
                        )
                        b0, b1 = plsc.unpack(
                            xr_[e, pl.ds(lh * 32, 32)],
                            format=plsc.PackFormat.INTERLEAVED,
                        )
                        t0 = a0 + b0
                        t1 = a1 + b1
                        l0 = jnp.maximum(t0, 0.2 * t0)
                        l1 = jnp.maximum(t1, 0.2 * t1)
                        s = l0 * att_r[2 * lh] + l1 * att_r[2 * lh + 1]
                        sc = jnp.sum(s)
                        wv = jnp.exp(jnp.full((16,), sc, jnp.float32))
                        o_[e, pl.ds(lh * 32, 16)] = a0 * wv
                        o_[e, pl.ds(lh * 32 + 16, 16)] = a1 * wv
                        wrow = jnp.where(lane == lh, wv, wrow)
                    o_[e, pl.ds(HALF, 16)] = wrow

                pltpu.async_copy(o_, table.at[didx.at[j]], ssem, add=True)

            def pair_body(jj, carry):
                j = 2 * jj
                fire(1, j + 1)
                drain_gather(0, j)
                compute(0, j, jj > 0)

                @pl.when(j + 2 < ssc)
                def _():
                    fire(0, j + 2)

                drain_gather(1, j + 1)
                compute(1, j + 1, jj > 0)
                return carry

            for sci in range(3):
                row0 = sid * chunks_per_sub + sci * ssc
                pltpu.sync_copy(src_hbm.at[pl.ds(row0, ssc)], sidx)
                pltpu.sync_copy(dst_hbm.at[pl.ds(row0, ssc)], didx)
                fire(0, 0)
                lax.fori_loop(0, ssc // 2, pair_body, 0)
                pltpu.make_async_copy(orow0, table.at[didx.at[0]], scsem0).wait()
                pltpu.make_async_copy(orow1, table.at[didx.at[0]], scsem1).wait()

        pl.when(cid == 0)(lambda: sweep(0, xl0_hbm, xr0_hbm))
        pl.when(cid == 1)(lambda: sweep(1, xl1_hbm, xr1_hbm))
        plsc.subcore_barrier()

        def copy_out(cc):
            for k in range(ROWS_PER_SUB // EB):
                r0 = sid * ROWS_PER_SUB + k * EB
                pltpu.sync_copy(table.at[pl.ds(r0, EB)], orow0)
                pltpu.sync_copy(orow0, out_hbm.at[cc, pl.ds(r0, EB)])

        pl.when(cid == 0)(lambda: copy_out(0))
        pl.when(cid == 1)(lambda: copy_out(1))

    return edge_kernel


def _gate_body(
    num_ref, h_ref, exp_ref, bias_ref, w1a_ref, w1b_ref, b1_ref,
    w2a_ref, w2b_ref, b2_ref, out_ref,
):
    n0 = num_ref[0]
    n1 = num_ref[1]
    agg = jnp.concatenate([n0[:, :HALF], n1[:, :HALF]], axis=1)
    den4 = jnp.concatenate([n0[:, HALF:HALF + 2], n1[:, HALF:HALF + 2]], axis=1)
    den = jnp.dot(den4, exp_ref[...], preferred_element_type=jnp.float32)
    f = jax.nn.sigmoid(agg / (den + 1e-16) + bias_ref[...])
    hb = h_ref[...]
    ru = jax.nn.sigmoid(
        jnp.dot(f, w1a_ref[...], preferred_element_type=jnp.float32)
        + jnp.dot(hb, w1b_ref[...], preferred_element_type=jnp.float32)
        + b1_ref[...]
    )
    r = ru[:, :OUT]
    u = ru[:, OUT:]
    cc = jnp.tanh(
        jnp.dot(f, w2a_ref[...], preferred_element_type=jnp.float32)
        + jnp.dot(r * hb, w2b_ref[...], preferred_element_type=jnp.float32)
        + b2_ref[...]
    )
    out_ref[...] = u * hb + (1.0 - u) * cc


def _gate(num2, h, bias, W1, b1, W2, b2):
    blk = 1000
    grid = N // blk
    j = jnp.arange(OUT, dtype=jnp.int32)
    jj = j % 64
    lh = jj // 32
    g = 2 * (j // 64) + lh
    r = jj % 32
    perm = jnp.where(r < 16, 32 * g + 2 * r, 32 * g + 2 * (r - 16) + 1)
    expand = (g[None, :] == jnp.arange(4, dtype=jnp.int32)[:, None]).astype(
        jnp.float32
    )
    bias = bias[perm]
    W1a = W1[:OUT][perm]
    W2a = W2[:OUT][perm]
    return pl.pallas_call(
        _gate_body,
        grid=(grid,),
        in_specs=[
            pl.BlockSpec((2, blk, WREC), lambda i: (0, i, 0)),
            pl.BlockSpec((blk, OUT), lambda i: (i, 0)),
            pl.BlockSpec((4, OUT), lambda i: (0, 0)),
            pl.BlockSpec((1, OUT), lambda i: (0, 0)),
            pl.BlockSpec((OUT, 2 * OUT), lambda i: (0, 0)),
            pl.BlockSpec((OUT, 2 * OUT), lambda i: (0, 0)),
            pl.BlockSpec((1, 2 * OUT), lambda i: (0, 0)),
            pl.BlockSpec((OUT, OUT), lambda i: (0, 0)),
            pl.BlockSpec((OUT, OUT), lambda i: (0, 0)),
            pl.BlockSpec((1, OUT), lambda i: (0, 0)),
        ],
        out_specs=pl.BlockSpec((blk, OUT), lambda i: (i, 0)),
        out_shape=jax.ShapeDtypeStruct((N, OUT), jnp.float32),
    )(
        num2, h, expand, bias.reshape(1, OUT),
        W1a, W1[OUT:], b1.reshape(1, 2 * OUT),
        W2a, W2[OUT:], b2.reshape(1, OUT),
    )


def kernel(x, edge_index, edge_weight, h, Wl, bl, Wr, br, att, bias, W1, b1, W2, b2):
    del edge_weight

    ne = edge_index.shape[1] + N
    q = 6 * NSUB * EB
    ep = ((ne + q - 1) // q) * q
    chunks_per_sub = ep // (NSUB * EB)

    loops = jnp.arange(N, dtype=jnp.int32)
    padv = jnp.full((ep - ne,), N, dtype=jnp.int32)
    src = jnp.concatenate(
        [edge_index[0].astype(jnp.int32), loops, padv]
    ).reshape(-1, EB)
    dst = jnp.concatenate(
        [edge_index[1].astype(jnp.int32), loops, padv]
    ).reshape(-1, EB)

    xl0, xl1, xr0, xr1 = _project(x, Wl, bl, Wr, br)

    att2 = jnp.stack([att[:, 0::2], att[:, 1::2]], axis=1).reshape(8, 16)
    num2 = _make_edge_kernel(chunks_per_sub)(xl0, xl1, xr0, xr1, src, dst, att2)

    return _gate(num2, h, bias, W1, b1, W2, b2)
